# baseline (speedup 1.0000x reference)
import functools

import jax
import jax.numpy as jnp
from jax import lax
from jax.experimental import pallas as pl
from jax.experimental.pallas import tpu as pltpu
from jax.experimental.pallas import tpu_sc as plsc

B = 4096
CTX = 20
D = 128
V = 100000

NC = 2
NS = 16
NW = NC * NS
BPW = B // NW
CH = 4
NCHUNK = BPW // CH
IPC = CH * CTX


def _gather_mean_sc(idx_flat, emb):
  mesh = plsc.VectorSubcoreMesh(core_axis_name="c", subcore_axis_name="s")

  @functools.partial(
      pl.kernel,
      mesh=mesh,
      out_type=jax.ShapeDtypeStruct((B, D), jnp.float32),
      scratch_types=[
          pltpu.VMEM((IPC,), jnp.int32),
          pltpu.VMEM((IPC, D), jnp.float32),
          pltpu.VMEM((CH, D), jnp.float32),
          pltpu.SemaphoreType.DMA,
      ],
  )
  def k(idx_hbm, emb_hbm, ctx_hbm, idx_v, rows_v, acc_v, sem):
    wid = lax.axis_index("s") * NC + lax.axis_index("c")

    def chunk(kk, _):
      row0 = wid * BPW + kk * CH
      pltpu.sync_copy(idx_hbm.at[pl.ds(row0 * CTX, IPC)], idx_v)
      pltpu.async_copy(emb_hbm.at[idx_v], rows_v, sem).wait()

      def per_row(r, _):
        base = r * CTX
        for dd in range(D // 16):
          sl = pl.ds(dd * 16, 16)
          a = rows_v[base, sl]
          for ci in range(1, CTX):
            a = a + rows_v[base + ci, sl]
          acc_v[r, sl] = a * (1.0 / CTX)
        return 0

      lax.fori_loop(0, CH, per_row, 0)
      pltpu.sync_copy(acc_v, ctx_hbm.at[pl.ds(row0, CH)])
      return 0

    lax.fori_loop(0, NCHUNK, chunk, 0)

  return k(idx_flat, emb)


BN = 512
GRID_N = pl.cdiv(V, BN)


def _proj_kernel(ctx_ref, w_ref, b_ref, out_ref):
  c = ctx_ref[...].astype(jnp.bfloat16)
  w = w_ref[...].astype(jnp.bfloat16)
  acc = lax.dot_general(c, w, (((1,), (1,)), ((), ())),
                        preferred_element_type=jnp.float32)
  out_ref[...] = acc + b_ref[...]


def _project(ctx, W, b2):
  return pl.pallas_call(
      _proj_kernel,
      grid=(GRID_N,),
      in_specs=[
          pl.BlockSpec((B, D), lambda n: (0, 0)),
          pl.BlockSpec((BN, D), lambda n: (n, 0)),
          pl.BlockSpec((1, BN), lambda n: (0, n)),
      ],
      out_specs=pl.BlockSpec((B, BN), lambda n: (0, n)),
      out_shape=jax.ShapeDtypeStruct((B, V), jnp.float32),
      compiler_params=pltpu.CompilerParams(
          dimension_semantics=("parallel",)),
  )(ctx, W, b2)


def kernel(X, emb, W, b):
  idx = X.astype(jnp.int32).reshape(-1)
  ctx = _gather_mean_sc(idx, emb)
  return _project(ctx, W, b.reshape(1, V))
